# baseline (speedup 1.0000x reference)
import functools

import jax
import jax.numpy as jnp
from jax import lax
from jax.experimental import pallas as pl
from jax.experimental.pallas import tpu as pltpu
from jax.experimental.pallas import tpu_sc as plsc

NC = 2
NS = 16
NT = NC * NS
CH = 128
RB = 512


def _sc_segment_sum(y, idx3d, n_pad, ch, cpt0, cpt1):
    d = y.shape[1]
    assert cpt0 % 4 == 0 and cpt1 % 4 == 0 and cpt0 >= 4 and cpt1 >= 4
    assert idx3d.shape[0] == NS * (cpt0 + cpt1)
    rpt = n_pad // NS
    zeros = jnp.zeros((n_pad, d), jnp.float32)

    mesh = plsc.VectorSubcoreMesh(core_axis_name="c", subcore_axis_name="s")

    @functools.partial(
        pl.kernel,
        out_type=jax.ShapeDtypeStruct((NC * n_pad, d), jnp.float32),
        mesh=mesh,
        compiler_params=pltpu.CompilerParams(use_tc_tiling_on_sc=False),
        scratch_types=(
            [pltpu.VMEM((2, ch), jnp.int32) for _ in range(4)]
            + [pltpu.VMEM((ch, d), jnp.float32) for _ in range(4)]
            + [pltpu.VMEM_SHARED((n_pad, d), jnp.float32)]
            + [pltpu.SemaphoreType.DMA] * 12
        ),
    )
    def run(y_hbm, idx_hbm, z_hbm, out_hbm,
            ib0, ib1, ib2, ib3, db0, db1, db2, db3, accum,
            sl0, sl1, sl2, sl3, sg0, sg1, sg2, sg3, ss0, ss1, ss2, ss3):
        ib = [ib0, ib1, ib2, ib3]
        db = [db0, db1, db2, db3]
        sl = [sl0, sl1, sl2, sl3]
        sg = [sg0, sg1, sg2, sg3]
        ss = [ss0, ss1, ss2, ss3]

        c = lax.axis_index("c")
        s = lax.axis_index("s")
        tot = jnp.where(c == 0, cpt0, cpt1)
        base = jnp.where(c == 0, s * cpt0, NS * cpt0 + s * cpt1)

        pltpu.sync_copy(z_hbm.at[pl.ds(s * rpt, rpt)],
                        accum.at[pl.ds(s * rpt, rpt)])
        plsc.subcore_barrier()

        def load(i, k):
            pltpu.async_copy(idx_hbm.at[base + i], ib[k], sl[k])

        def load_wait(k):
            pltpu.make_async_copy(idx_hbm.at[0], ib[k], sl[k]).wait()

        def gather(i, k):
            del i
            pltpu.async_copy(y_hbm.at[ib[k].at[0]], db[k], sg[k])

        def gather_wait(k):
            pltpu.make_async_copy(y_hbm.at[ib[k].at[0]], db[k], sg[k]).wait()

        def scatter(i, k):
            del i
            pltpu.async_copy(db[k], accum.at[ib[k].at[1]], ss[k], add=True)

        def scatter_wait(k):
            pltpu.make_async_copy(db[k], accum.at[ib[k].at[1]], ss[k]).wait()

        load(0, 0)
        load(1, 1)
        load_wait(0)
        gather(0, 0)
        load(2, 2)
        load_wait(1)
        gather(1, 1)
        gather_wait(0)
        scatter(0, 0)
        load(3, 3)

        @pl.loop(0, (tot - 4) // 4)
        def _(q):
            i0 = 4 * q + 2
            for k in range(4):
                b, pr, f = (2 + k) % 4, (1 + k) % 4, k % 4
                load_wait(b)
                gather(i0 + k, b)
                gather_wait(pr)
                scatter(i0 + k - 1, pr)
                scatter_wait(f)
                load(i0 + k + 2, f)

        load_wait(2)
        gather(tot - 2, 2)
        gather_wait(1)
        scatter(tot - 3, 1)
        load_wait(3)
        gather(tot - 1, 3)
        gather_wait(2)
        scatter(tot - 2, 2)
        gather_wait(3)
        scatter(tot - 1, 3)
        for k in range(4):
            scatter_wait(k)

        plsc.subcore_barrier()
        pltpu.sync_copy(
            accum.at[pl.ds(s * rpt, rpt)],
            out_hbm.at[pl.ds(c * n_pad + s * rpt, rpt)],
        )

    return run(y, idx3d, zeros)


def _dinv(d0, d1):
    return lax.rsqrt(1.0 + d0[:, 0:1] + d1[:, 0:1])


def _mm_body(x_ref, w_ref, o_ref):
    o_ref[...] = jnp.dot(x_ref[...], w_ref[...],
                         preferred_element_type=jnp.float32,
                         precision=lax.Precision.HIGHEST)


def _tc_matmul(x_pad, w):
    n_pad, k = x_pad.shape
    m = w.shape[1]
    return pl.pallas_call(
        _mm_body,
        grid=(n_pad // RB,),
        in_specs=[pl.BlockSpec((RB, k), lambda i: (i, 0)),
                  pl.BlockSpec((k, m), lambda i: (0, 0))],
        out_specs=pl.BlockSpec((RB, m), lambda i: (i, 0)),
        out_shape=jax.ShapeDtypeStruct((n_pad, m), jnp.float32),
    )(x_pad, w)


def _scale_body(d0_ref, d1_ref, xw_ref, y_ref):
    y_ref[...] = xw_ref[...] * _dinv(d0_ref[...], d1_ref[...])


def _tc_scale(deg_p, xw):
    n_pad, dh = xw.shape
    nb = n_pad // RB
    return pl.pallas_call(
        _scale_body,
        grid=(nb,),
        in_specs=[pl.BlockSpec((RB, 16), lambda i: (i, 0)),
                  pl.BlockSpec((RB, 16), lambda i: (i + nb, 0)),
                  pl.BlockSpec((RB, dh), lambda i: (i, 0))],
        out_specs=pl.BlockSpec((RB, dh), lambda i: (i, 0)),
        out_shape=jax.ShapeDtypeStruct((n_pad, dh), jnp.float32),
    )(deg_p, deg_p, xw)


def _mid_body(d0, d1, a0, a1, xw1, b1, w2, xw2_o, y2_o):
    dinv = _dinv(d0[...], d1[...])
    h = (a0[...] + a1[...]) * dinv + xw1[...] * (dinv * dinv) + b1[...]
    h = jnp.maximum(h, 0.0)
    xw2 = jnp.dot(h, w2[...], preferred_element_type=jnp.float32,
                  precision=lax.Precision.HIGHEST)
    xw2_o[...] = xw2
    y2_o[...] = xw2 * dinv


def _tc_mid(deg_p, agg1_p, xw1, b1r, w2):
    n_pad, dh = xw1.shape
    do = w2.shape[1]
    nb = n_pad // RB
    return pl.pallas_call(
        _mid_body,
        grid=(nb,),
        in_specs=[pl.BlockSpec((RB, 16), lambda i: (i, 0)),
                  pl.BlockSpec((RB, 16), lambda i: (i + nb, 0)),
                  pl.BlockSpec((RB, dh), lambda i: (i, 0)),
                  pl.BlockSpec((RB, dh), lambda i: (i + nb, 0)),
                  pl.BlockSpec((RB, dh), lambda i: (i, 0)),
                  pl.BlockSpec((1, dh), lambda i: (0, 0)),
                  pl.BlockSpec((dh, do), lambda i: (0, 0))],
        out_specs=[pl.BlockSpec((RB, do), lambda i: (i, 0)),
                   pl.BlockSpec((RB, do), lambda i: (i, 0))],
        out_shape=[jax.ShapeDtypeStruct((n_pad, do), jnp.float32),
                   jax.ShapeDtypeStruct((n_pad, do), jnp.float32)],
    )(deg_p, deg_p, agg1_p, agg1_p, xw1, b1r, w2)


def _final_body(d0, d1, g0, g1, xw2, b2, o_ref):
    dinv = _dinv(d0[...], d1[...])
    o_ref[...] = (g0[...] + g1[...]) * dinv + xw2[...] * (dinv * dinv) + b2[...]


def _tc_final(deg_p, agg2_p, xw2, b2r):
    n_pad, do = xw2.shape
    nb = n_pad // RB
    return pl.pallas_call(
        _final_body,
        grid=(nb,),
        in_specs=[pl.BlockSpec((RB, 16), lambda i: (i, 0)),
                  pl.BlockSpec((RB, 16), lambda i: (i + nb, 0)),
                  pl.BlockSpec((RB, do), lambda i: (i, 0)),
                  pl.BlockSpec((RB, do), lambda i: (i + nb, 0)),
                  pl.BlockSpec((RB, do), lambda i: (i, 0)),
                  pl.BlockSpec((1, do), lambda i: (0, 0))],
        out_specs=pl.BlockSpec((RB, do), lambda i: (i, 0)),
        out_shape=jax.ShapeDtypeStruct((n_pad, do), jnp.float32),
    )(deg_p, deg_p, agg2_p, agg2_p, xw2, b2r)


def kernel(x, edge_index, W1, b1, W2, b2):
    n, d_in = x.shape
    e = edge_index.shape[1]

    blk = NS * CH
    n_pad = ((n + 1 + blk - 1) // blk) * blk

    def make_idx(ch, cpt0, cpt1):
        n_chunks = NS * (cpt0 + cpt1)
        e_pad = n_chunks * ch
        pad = jnp.full((e_pad - e,), n, dtype=jnp.int32)
        s2 = jnp.concatenate([edge_index[0], pad]).reshape(n_chunks, 1, ch)
        d2 = jnp.concatenate([edge_index[1], pad]).reshape(n_chunks, 1, ch)
        return jnp.concatenate([s2, d2], axis=1)

    C0_DEG, C1_DEG = 96, 64
    C0_A2, C1_A2 = 128, 32
    C0_64, C1_64 = 312, 4
    idx128 = make_idx(CH, C0_DEG, C1_DEG)
    idx64 = make_idx(64, C0_64, C1_64)

    x_pad = jnp.pad(x, ((0, n_pad - n), (0, 0)))
    ones16 = jnp.ones((n_pad, 16), jnp.float32)

    deg_p = _sc_segment_sum(ones16, idx128, n_pad, CH, C0_DEG, C1_DEG)
    xw1 = _tc_matmul(x_pad, W1)
    y1 = _tc_scale(deg_p, xw1)
    agg1_p = _sc_segment_sum(y1, idx64, n_pad, 64, C0_64, C1_64)
    xw2, y2 = _tc_mid(deg_p, agg1_p, xw1, b1.reshape(1, -1), W2)
    agg2_p = _sc_segment_sum(y2, idx128, n_pad, CH, C0_A2, C1_A2)
    out = _tc_final(deg_p, agg2_p, xw2, b2.reshape(1, -1))
    return out[:n]


# speedup vs baseline: 1.1449x; 1.1449x over previous
import functools

import jax
import jax.numpy as jnp
from jax import lax
from jax.experimental import pallas as pl
from jax.experimental.pallas import tpu as pltpu
from jax.experimental.pallas import tpu_sc as plsc

NC = 2
NS = 16
NT = NC * NS
CH = 128
RB = 512


def _sc_segment_sum(y, idx3d, n_pad, ch, cpt0, cpt1):
    d = y.shape[1]
    assert cpt0 % 4 == 0 and cpt1 % 4 == 0 and cpt0 >= 4 and cpt1 >= 4
    assert idx3d.shape[0] == NS * (cpt0 + cpt1)
    rpt = n_pad // NS
    zeros = jnp.zeros((n_pad, d), jnp.float32)

    mesh = plsc.VectorSubcoreMesh(core_axis_name="c", subcore_axis_name="s")

    @functools.partial(
        pl.kernel,
        out_type=jax.ShapeDtypeStruct((NC * n_pad, d), jnp.float32),
        mesh=mesh,
        compiler_params=pltpu.CompilerParams(use_tc_tiling_on_sc=False),
        scratch_types=(
            [pltpu.VMEM((2, ch), jnp.int32) for _ in range(4)]
            + [pltpu.VMEM((ch, d), jnp.float32) for _ in range(4)]
            + [pltpu.VMEM_SHARED((n_pad, d), jnp.float32)]
            + [pltpu.SemaphoreType.DMA] * 12
        ),
    )
    def run(y_hbm, idx_hbm, z_hbm, out_hbm,
            ib0, ib1, ib2, ib3, db0, db1, db2, db3, accum,
            sl0, sl1, sl2, sl3, sg0, sg1, sg2, sg3, ss0, ss1, ss2, ss3):
        ib = [ib0, ib1, ib2, ib3]
        db = [db0, db1, db2, db3]
        sl = [sl0, sl1, sl2, sl3]
        sg = [sg0, sg1, sg2, sg3]
        ss = [ss0, ss1, ss2, ss3]

        c = lax.axis_index("c")
        s = lax.axis_index("s")
        tot = jnp.where(c == 0, cpt0, cpt1)
        base = jnp.where(c == 0, s * cpt0, NS * cpt0 + s * cpt1)

        pltpu.sync_copy(z_hbm.at[pl.ds(s * rpt, rpt)],
                        accum.at[pl.ds(s * rpt, rpt)])
        plsc.subcore_barrier()

        def load(i, k):
            pltpu.async_copy(idx_hbm.at[base + i], ib[k], sl[k])

        def load_wait(k):
            pltpu.make_async_copy(idx_hbm.at[0], ib[k], sl[k]).wait()

        def gather(i, k):
            del i
            pltpu.async_copy(y_hbm.at[ib[k].at[0]], db[k], sg[k])

        def gather_wait(k):
            pltpu.make_async_copy(y_hbm.at[ib[k].at[0]], db[k], sg[k]).wait()

        def scatter(i, k):
            del i
            pltpu.async_copy(db[k], accum.at[ib[k].at[1]], ss[k], add=True)

        def scatter_wait(k):
            pltpu.make_async_copy(db[k], accum.at[ib[k].at[1]], ss[k]).wait()

        load(0, 0)
        load(1, 1)
        load_wait(0)
        gather(0, 0)
        load(2, 2)
        load_wait(1)
        gather(1, 1)
        gather_wait(0)
        scatter(0, 0)
        load(3, 3)

        @pl.loop(0, (tot - 4) // 4)
        def _(q):
            i0 = 4 * q + 2
            for k in range(4):
                b, pr, f = (2 + k) % 4, (1 + k) % 4, k % 4
                load_wait(b)
                gather(i0 + k, b)
                gather_wait(pr)
                scatter(i0 + k - 1, pr)
                scatter_wait(f)
                load(i0 + k + 2, f)

        load_wait(2)
        gather(tot - 2, 2)
        gather_wait(1)
        scatter(tot - 3, 1)
        load_wait(3)
        gather(tot - 1, 3)
        gather_wait(2)
        scatter(tot - 2, 2)
        gather_wait(3)
        scatter(tot - 1, 3)
        for k in range(4):
            scatter_wait(k)

        plsc.subcore_barrier()
        pltpu.sync_copy(
            accum.at[pl.ds(s * rpt, rpt)],
            out_hbm.at[pl.ds(c * n_pad + s * rpt, rpt)],
        )

    return run(y, idx3d, zeros)


def _sc_degree(idx3d, n_pad, cpt0, cpt1):
    d = 16
    assert cpt0 % 4 == 0 and cpt1 % 4 == 0 and cpt0 >= 4 and cpt1 >= 4
    ch = idx3d.shape[2]
    rpt = n_pad // NS
    zeros = jnp.zeros((n_pad, d), jnp.float32)

    mesh = plsc.VectorSubcoreMesh(core_axis_name="c", subcore_axis_name="s")

    @functools.partial(
        pl.kernel,
        out_type=jax.ShapeDtypeStruct((NC * n_pad, d), jnp.float32),
        mesh=mesh,
        compiler_params=pltpu.CompilerParams(use_tc_tiling_on_sc=False),
        scratch_types=(
            [pltpu.VMEM((2, ch), jnp.int32) for _ in range(4)]
            + [pltpu.VMEM((ch, d), jnp.float32)]
            + [pltpu.VMEM_SHARED((n_pad, d), jnp.float32)]
            + [pltpu.SemaphoreType.DMA] * 8
        ),
    )
    def run(idx_hbm, z_hbm, out_hbm, ib0, ib1, ib2, ib3, ones, accum,
            sl0, sl1, sl2, sl3, ss0, ss1, ss2, ss3):
        ib = [ib0, ib1, ib2, ib3]
        sl = [sl0, sl1, sl2, sl3]
        ss = [ss0, ss1, ss2, ss3]

        c = lax.axis_index("c")
        s = lax.axis_index("s")
        tot = jnp.where(c == 0, cpt0, cpt1)
        base = jnp.where(c == 0, s * cpt0, NS * cpt0 + s * cpt1)

        @pl.loop(0, ch)
        def _(r):
            ones[r, pl.ds(0, 16)] = jnp.ones((16,), jnp.float32)

        pltpu.sync_copy(z_hbm.at[pl.ds(s * rpt, rpt)],
                        accum.at[pl.ds(s * rpt, rpt)])
        plsc.subcore_barrier()

        def load(i, k):
            pltpu.async_copy(idx_hbm.at[base + i], ib[k], sl[k])

        def load_wait(k):
            pltpu.make_async_copy(idx_hbm.at[0], ib[k], sl[k]).wait()

        def scatter(k):
            pltpu.async_copy(ones, accum.at[ib[k].at[1]], ss[k], add=True)

        def scatter_wait(k):
            pltpu.make_async_copy(ones, accum.at[ib[k].at[1]], ss[k]).wait()

        load(0, 0)
        load(1, 1)
        load_wait(0)
        scatter(0)
        load(2, 2)
        load_wait(1)
        scatter(1)
        load(3, 3)

        @pl.loop(0, (tot - 4) // 4)
        def _(q):
            i0 = 4 * q + 2
            for k in range(4):
                b, f = (2 + k) % 4, k % 4
                load_wait(b)
                scatter(b)
                scatter_wait(f)
                load(i0 + k + 2, f)

        load_wait(2)
        scatter(2)
        load_wait(3)
        scatter(3)
        for k in range(4):
            scatter_wait(k)

        plsc.subcore_barrier()
        pltpu.sync_copy(
            accum.at[pl.ds(s * rpt, rpt)],
            out_hbm.at[pl.ds(c * n_pad + s * rpt, rpt)],
        )

    return run(idx3d, zeros)


def _dinv(d0, d1):
    return lax.rsqrt(1.0 + d0[:, 0:1] + d1[:, 0:1])


def _mm_body(x_ref, w_ref, o_ref):
    o_ref[...] = jnp.dot(x_ref[...], w_ref[...],
                         preferred_element_type=jnp.float32,
                         precision=lax.Precision.HIGHEST)


def _tc_matmul(x_pad, w):
    n_pad, k = x_pad.shape
    m = w.shape[1]
    return pl.pallas_call(
        _mm_body,
        grid=(n_pad // RB,),
        in_specs=[pl.BlockSpec((RB, k), lambda i: (i, 0)),
                  pl.BlockSpec((k, m), lambda i: (0, 0))],
        out_specs=pl.BlockSpec((RB, m), lambda i: (i, 0)),
        out_shape=jax.ShapeDtypeStruct((n_pad, m), jnp.float32),
    )(x_pad, w)


def _scale_body(d0_ref, d1_ref, xw_ref, y_ref):
    y_ref[...] = xw_ref[...] * _dinv(d0_ref[...], d1_ref[...])


def _tc_scale(deg_p, xw):
    n_pad, dh = xw.shape
    nb = n_pad // RB
    return pl.pallas_call(
        _scale_body,
        grid=(nb,),
        in_specs=[pl.BlockSpec((RB, 16), lambda i: (i, 0)),
                  pl.BlockSpec((RB, 16), lambda i: (i + nb, 0)),
                  pl.BlockSpec((RB, dh), lambda i: (i, 0))],
        out_specs=pl.BlockSpec((RB, dh), lambda i: (i, 0)),
        out_shape=jax.ShapeDtypeStruct((n_pad, dh), jnp.float32),
    )(deg_p, deg_p, xw)


def _mid_body(d0, d1, a0, a1, xw1, b1, w2, xw2_o, y2_o):
    dinv = _dinv(d0[...], d1[...])
    h = (a0[...] + a1[...]) * dinv + xw1[...] * (dinv * dinv) + b1[...]
    h = jnp.maximum(h, 0.0)
    xw2 = jnp.dot(h, w2[...], preferred_element_type=jnp.float32,
                  precision=lax.Precision.HIGHEST)
    xw2_o[...] = xw2
    y2_o[...] = xw2 * dinv


def _tc_mid(deg_p, agg1_p, xw1, b1r, w2):
    n_pad, dh = xw1.shape
    do = w2.shape[1]
    nb = n_pad // RB
    return pl.pallas_call(
        _mid_body,
        grid=(nb,),
        in_specs=[pl.BlockSpec((RB, 16), lambda i: (i, 0)),
                  pl.BlockSpec((RB, 16), lambda i: (i + nb, 0)),
                  pl.BlockSpec((RB, dh), lambda i: (i, 0)),
                  pl.BlockSpec((RB, dh), lambda i: (i + nb, 0)),
                  pl.BlockSpec((RB, dh), lambda i: (i, 0)),
                  pl.BlockSpec((1, dh), lambda i: (0, 0)),
                  pl.BlockSpec((dh, do), lambda i: (0, 0))],
        out_specs=[pl.BlockSpec((RB, do), lambda i: (i, 0)),
                   pl.BlockSpec((RB, do), lambda i: (i, 0))],
        out_shape=[jax.ShapeDtypeStruct((n_pad, do), jnp.float32),
                   jax.ShapeDtypeStruct((n_pad, do), jnp.float32)],
    )(deg_p, deg_p, agg1_p, agg1_p, xw1, b1r, w2)


def _final_body(d0, d1, g0, g1, xw2, b2, o_ref):
    dinv = _dinv(d0[...], d1[...])
    o_ref[...] = (g0[...] + g1[...]) * dinv + xw2[...] * (dinv * dinv) + b2[...]


def _tc_final(deg_p, agg2_p, xw2, b2r):
    n_pad, do = xw2.shape
    nb = n_pad // RB
    return pl.pallas_call(
        _final_body,
        grid=(nb,),
        in_specs=[pl.BlockSpec((RB, 16), lambda i: (i, 0)),
                  pl.BlockSpec((RB, 16), lambda i: (i + nb, 0)),
                  pl.BlockSpec((RB, do), lambda i: (i, 0)),
                  pl.BlockSpec((RB, do), lambda i: (i + nb, 0)),
                  pl.BlockSpec((RB, do), lambda i: (i, 0)),
                  pl.BlockSpec((1, do), lambda i: (0, 0))],
        out_specs=pl.BlockSpec((RB, do), lambda i: (i, 0)),
        out_shape=jax.ShapeDtypeStruct((n_pad, do), jnp.float32),
    )(deg_p, deg_p, agg2_p, agg2_p, xw2, b2r)


def kernel(x, edge_index, W1, b1, W2, b2):
    n, d_in = x.shape
    e = edge_index.shape[1]

    blk = NS * CH
    n_pad = ((n + 1 + blk - 1) // blk) * blk

    def make_idx(ch, cpt0, cpt1):
        n_chunks = NS * (cpt0 + cpt1)
        e_pad = n_chunks * ch
        pad = jnp.full((e_pad - e,), n, dtype=jnp.int32)
        s2 = jnp.concatenate([edge_index[0], pad]).reshape(n_chunks, 1, ch)
        d2 = jnp.concatenate([edge_index[1], pad]).reshape(n_chunks, 1, ch)
        return jnp.concatenate([s2, d2], axis=1)

    C0_DEG, C1_DEG = 96, 64
    C0_A2, C1_A2 = 128, 32
    C0_64, C1_64 = 300, 16
    idx128 = make_idx(CH, C0_DEG, C1_DEG)
    idx64 = make_idx(64, C0_64, C1_64)

    x_pad = jnp.pad(x, ((0, n_pad - n), (0, 0)))
    deg_p = _sc_degree(idx128, n_pad, C0_DEG, C1_DEG)
    xw1 = _tc_matmul(x_pad, W1)
    y1 = _tc_scale(deg_p, xw1)
    agg1_p = _sc_segment_sum(y1, idx64, n_pad, 64, C0_64, C1_64)
    xw2, y2 = _tc_mid(deg_p, agg1_p, xw1, b1.reshape(1, -1), W2)
    agg2_p = _sc_segment_sum(y2, idx128, n_pad, CH, C0_A2, C1_A2)
    out = _tc_final(deg_p, agg2_p, xw2, b2.reshape(1, -1))
    return out[:n]


# speedup vs baseline: 1.1505x; 1.0049x over previous
import functools

import jax
import jax.numpy as jnp
from jax import lax
from jax.experimental import pallas as pl
from jax.experimental.pallas import tpu as pltpu
from jax.experimental.pallas import tpu_sc as plsc

NC = 2
NS = 16
NT = NC * NS
CH = 128
RB = 1024


def _sc_segment_sum(y, idx3d, n_pad, ch, cpt0, cpt1):
    d = y.shape[1]
    assert cpt0 % 4 == 0 and cpt1 % 4 == 0 and cpt0 >= 4 and cpt1 >= 4
    assert idx3d.shape[0] == NS * (cpt0 + cpt1)
    rpt = n_pad // NS
    zeros = jnp.zeros((n_pad, d), jnp.float32)

    mesh = plsc.VectorSubcoreMesh(core_axis_name="c", subcore_axis_name="s")

    @functools.partial(
        pl.kernel,
        out_type=jax.ShapeDtypeStruct((NC * n_pad, d), jnp.float32),
        mesh=mesh,
        compiler_params=pltpu.CompilerParams(use_tc_tiling_on_sc=False),
        scratch_types=(
            [pltpu.VMEM((2, ch), jnp.int32) for _ in range(4)]
            + [pltpu.VMEM((ch, d), jnp.float32) for _ in range(4)]
            + [pltpu.VMEM_SHARED((n_pad, d), jnp.float32)]
            + [pltpu.SemaphoreType.DMA] * 12
        ),
    )
    def run(y_hbm, idx_hbm, z_hbm, out_hbm,
            ib0, ib1, ib2, ib3, db0, db1, db2, db3, accum,
            sl0, sl1, sl2, sl3, sg0, sg1, sg2, sg3, ss0, ss1, ss2, ss3):
        ib = [ib0, ib1, ib2, ib3]
        db = [db0, db1, db2, db3]
        sl = [sl0, sl1, sl2, sl3]
        sg = [sg0, sg1, sg2, sg3]
        ss = [ss0, ss1, ss2, ss3]

        c = lax.axis_index("c")
        s = lax.axis_index("s")
        tot = jnp.where(c == 0, cpt0, cpt1)
        base = jnp.where(c == 0, s * cpt0, NS * cpt0 + s * cpt1)

        pltpu.sync_copy(z_hbm.at[pl.ds(s * rpt, rpt)],
                        accum.at[pl.ds(s * rpt, rpt)])
        plsc.subcore_barrier()

        def load(i, k):
            pltpu.async_copy(idx_hbm.at[base + i], ib[k], sl[k])

        def load_wait(k):
            pltpu.make_async_copy(idx_hbm.at[0], ib[k], sl[k]).wait()

        def gather(i, k):
            del i
            pltpu.async_copy(y_hbm.at[ib[k].at[0]], db[k], sg[k])

        def gather_wait(k):
            pltpu.make_async_copy(y_hbm.at[ib[k].at[0]], db[k], sg[k]).wait()

        def scatter(i, k):
            del i
            pltpu.async_copy(db[k], accum.at[ib[k].at[1]], ss[k], add=True)

        def scatter_wait(k):
            pltpu.make_async_copy(db[k], accum.at[ib[k].at[1]], ss[k]).wait()

        load(0, 0)
        load(1, 1)
        load_wait(0)
        gather(0, 0)
        load(2, 2)
        load_wait(1)
        gather(1, 1)
        gather_wait(0)
        scatter(0, 0)
        load(3, 3)

        @pl.loop(0, (tot - 4) // 4)
        def _(q):
            i0 = 4 * q + 2
            for k in range(4):
                b, pr, f = (2 + k) % 4, (1 + k) % 4, k % 4
                load_wait(b)
                gather(i0 + k, b)
                gather_wait(pr)
                scatter(i0 + k - 1, pr)
                scatter_wait(f)
                load(i0 + k + 2, f)

        load_wait(2)
        gather(tot - 2, 2)
        gather_wait(1)
        scatter(tot - 3, 1)
        load_wait(3)
        gather(tot - 1, 3)
        gather_wait(2)
        scatter(tot - 2, 2)
        gather_wait(3)
        scatter(tot - 1, 3)
        for k in range(4):
            scatter_wait(k)

        plsc.subcore_barrier()
        pltpu.sync_copy(
            accum.at[pl.ds(s * rpt, rpt)],
            out_hbm.at[pl.ds(c * n_pad + s * rpt, rpt)],
        )

    return run(y, idx3d, zeros)


def _sc_degree(idx3d, n_pad, cpt0, cpt1):
    d = 16
    assert cpt0 % 4 == 0 and cpt1 % 4 == 0 and cpt0 >= 4 and cpt1 >= 4
    ch = idx3d.shape[2]
    rpt = n_pad // NS
    zeros = jnp.zeros((n_pad, d), jnp.float32)

    mesh = plsc.VectorSubcoreMesh(core_axis_name="c", subcore_axis_name="s")

    @functools.partial(
        pl.kernel,
        out_type=jax.ShapeDtypeStruct((NC * n_pad, d), jnp.float32),
        mesh=mesh,
        compiler_params=pltpu.CompilerParams(use_tc_tiling_on_sc=False),
        scratch_types=(
            [pltpu.VMEM((2, ch), jnp.int32) for _ in range(4)]
            + [pltpu.VMEM((ch, d), jnp.float32)]
            + [pltpu.VMEM_SHARED((n_pad, d), jnp.float32)]
            + [pltpu.SemaphoreType.DMA] * 8
        ),
    )
    def run(idx_hbm, z_hbm, out_hbm, ib0, ib1, ib2, ib3, ones, accum,
            sl0, sl1, sl2, sl3, ss0, ss1, ss2, ss3):
        ib = [ib0, ib1, ib2, ib3]
        sl = [sl0, sl1, sl2, sl3]
        ss = [ss0, ss1, ss2, ss3]

        c = lax.axis_index("c")
        s = lax.axis_index("s")
        tot = jnp.where(c == 0, cpt0, cpt1)
        base = jnp.where(c == 0, s * cpt0, NS * cpt0 + s * cpt1)

        @pl.loop(0, ch)
        def _(r):
            ones[r, pl.ds(0, 16)] = jnp.ones((16,), jnp.float32)

        pltpu.sync_copy(z_hbm.at[pl.ds(s * rpt, rpt)],
                        accum.at[pl.ds(s * rpt, rpt)])
        plsc.subcore_barrier()

        def load(i, k):
            pltpu.async_copy(idx_hbm.at[base + i], ib[k], sl[k])

        def load_wait(k):
            pltpu.make_async_copy(idx_hbm.at[0], ib[k], sl[k]).wait()

        def scatter(k):
            pltpu.async_copy(ones, accum.at[ib[k].at[1]], ss[k], add=True)

        def scatter_wait(k):
            pltpu.make_async_copy(ones, accum.at[ib[k].at[1]], ss[k]).wait()

        load(0, 0)
        load(1, 1)
        load_wait(0)
        scatter(0)
        load(2, 2)
        load_wait(1)
        scatter(1)
        load(3, 3)

        @pl.loop(0, (tot - 4) // 4)
        def _(q):
            i0 = 4 * q + 2
            for k in range(4):
                b, f = (2 + k) % 4, k % 4
                load_wait(b)
                scatter(b)
                scatter_wait(f)
                load(i0 + k + 2, f)

        load_wait(2)
        scatter(2)
        load_wait(3)
        scatter(3)
        for k in range(4):
            scatter_wait(k)

        plsc.subcore_barrier()
        pltpu.sync_copy(
            accum.at[pl.ds(s * rpt, rpt)],
            out_hbm.at[pl.ds(c * n_pad + s * rpt, rpt)],
        )

    return run(idx3d, zeros)


def _dinv(d0, d1):
    return lax.rsqrt(1.0 + d0[:, 0:1] + d1[:, 0:1])


def _mm_scale_body(x_ref, w_ref, d0_ref, d1_ref, xw_ref, y_ref):
    xw = jnp.dot(x_ref[...], w_ref[...],
                 preferred_element_type=jnp.float32,
                 precision=lax.Precision.HIGHEST)
    xw_ref[...] = xw
    y_ref[...] = xw * _dinv(d0_ref[...], d1_ref[...])


def _tc_matmul_scale(x_pad, w, deg_p):
    n_pad, k = x_pad.shape
    m = w.shape[1]
    nb = n_pad // RB
    return pl.pallas_call(
        _mm_scale_body,
        grid=(nb,),
        in_specs=[pl.BlockSpec((RB, k), lambda i: (i, 0)),
                  pl.BlockSpec((k, m), lambda i: (0, 0)),
                  pl.BlockSpec((RB, 16), lambda i: (i, 0)),
                  pl.BlockSpec((RB, 16), lambda i: (i + nb, 0))],
        out_specs=[pl.BlockSpec((RB, m), lambda i: (i, 0)),
                   pl.BlockSpec((RB, m), lambda i: (i, 0))],
        out_shape=[jax.ShapeDtypeStruct((n_pad, m), jnp.float32),
                   jax.ShapeDtypeStruct((n_pad, m), jnp.float32)],
    )(x_pad, w, deg_p, deg_p)


def _mid_body(d0, d1, a0, a1, xw1, b1, w2, xw2_o, y2_o):
    dinv = _dinv(d0[...], d1[...])
    h = (a0[...] + a1[...]) * dinv + xw1[...] * (dinv * dinv) + b1[...]
    h = jnp.maximum(h, 0.0)
    xw2 = jnp.dot(h, w2[...], preferred_element_type=jnp.float32,
                  precision=lax.Precision.HIGHEST)
    xw2_o[...] = xw2
    y2_o[...] = xw2 * dinv


def _tc_mid(deg_p, agg1_p, xw1, b1r, w2):
    n_pad, dh = xw1.shape
    do = w2.shape[1]
    nb = n_pad // RB
    return pl.pallas_call(
        _mid_body,
        grid=(nb,),
        in_specs=[pl.BlockSpec((RB, 16), lambda i: (i, 0)),
                  pl.BlockSpec((RB, 16), lambda i: (i + nb, 0)),
                  pl.BlockSpec((RB, dh), lambda i: (i, 0)),
                  pl.BlockSpec((RB, dh), lambda i: (i + nb, 0)),
                  pl.BlockSpec((RB, dh), lambda i: (i, 0)),
                  pl.BlockSpec((1, dh), lambda i: (0, 0)),
                  pl.BlockSpec((dh, do), lambda i: (0, 0))],
        out_specs=[pl.BlockSpec((RB, do), lambda i: (i, 0)),
                   pl.BlockSpec((RB, do), lambda i: (i, 0))],
        out_shape=[jax.ShapeDtypeStruct((n_pad, do), jnp.float32),
                   jax.ShapeDtypeStruct((n_pad, do), jnp.float32)],
    )(deg_p, deg_p, agg1_p, agg1_p, xw1, b1r, w2)


def _final_body(d0, d1, g0, g1, xw2, b2, o_ref):
    dinv = _dinv(d0[...], d1[...])
    o_ref[...] = (g0[...] + g1[...]) * dinv + xw2[...] * (dinv * dinv) + b2[...]


def _tc_final(deg_p, agg2_p, xw2, b2r, n):
    n_pad, do = xw2.shape
    rb = 1000
    nb = n // rb
    return pl.pallas_call(
        _final_body,
        grid=(nb,),
        in_specs=[pl.BlockSpec((rb, 16), lambda i: (i, 0)),
                  pl.BlockSpec((rb, 16), lambda i: (i, 0)),
                  pl.BlockSpec((rb, do), lambda i: (i, 0)),
                  pl.BlockSpec((rb, do), lambda i: (i, 0)),
                  pl.BlockSpec((rb, do), lambda i: (i, 0)),
                  pl.BlockSpec((1, do), lambda i: (0, 0))],
        out_specs=pl.BlockSpec((rb, do), lambda i: (i, 0)),
        out_shape=jax.ShapeDtypeStruct((n, do), jnp.float32),
    )(deg_p[:n_pad], deg_p[n_pad:], agg2_p[:n_pad], agg2_p[n_pad:], xw2, b2r)


def kernel(x, edge_index, W1, b1, W2, b2):
    n, d_in = x.shape
    e = edge_index.shape[1]

    blk = NS * CH
    n_pad = ((n + 1 + blk - 1) // blk) * blk

    def make_idx(ch, cpt0, cpt1):
        n_chunks = NS * (cpt0 + cpt1)
        e_pad = n_chunks * ch
        pad = jnp.full((e_pad - e,), n, dtype=jnp.int32)
        s2 = jnp.concatenate([edge_index[0], pad]).reshape(n_chunks, 1, ch)
        d2 = jnp.concatenate([edge_index[1], pad]).reshape(n_chunks, 1, ch)
        return jnp.concatenate([s2, d2], axis=1)

    C0_DEG, C1_DEG = 96, 64
    C0_A2, C1_A2 = 128, 32
    C0_64, C1_64 = 300, 16
    idx128 = make_idx(CH, C0_DEG, C1_DEG)
    idx64 = make_idx(64, C0_64, C1_64)

    x_pad = jnp.pad(x, ((0, n_pad - n), (0, 0)))
    deg_p = _sc_degree(idx128, n_pad, C0_DEG, C1_DEG)
    xw1, y1 = _tc_matmul_scale(x_pad, W1, deg_p)
    agg1_p = _sc_segment_sum(y1, idx64, n_pad, 64, C0_64, C1_64)
    xw2, y2 = _tc_mid(deg_p, agg1_p, xw1, b1.reshape(1, -1), W2)
    agg2_p = _sc_segment_sum(y2, idx128, n_pad, CH, C0_A2, C1_A2)
    return _tc_final(deg_p, agg2_p, xw2, b2.reshape(1, -1), n)


# speedup vs baseline: 1.2211x; 1.0613x over previous
import functools

import jax
import jax.numpy as jnp
from jax import lax
from jax.experimental import pallas as pl
from jax.experimental.pallas import tpu as pltpu
from jax.experimental.pallas import tpu_sc as plsc

NC = 2
NS = 16
NT = NC * NS
CH = 128
RB = 1024


def _sc_segment_sum(y, idx3d, n_pad, ch, cpt0, cpt1):
    d = y.shape[1]
    assert cpt0 % 4 == 0 and cpt1 % 4 == 0 and cpt0 >= 4 and cpt1 >= 4
    assert idx3d.shape[0] == NS * (cpt0 + cpt1)
    rpt = n_pad // NS
    zeros = jnp.zeros((n_pad, d), jnp.float32)

    mesh = plsc.VectorSubcoreMesh(core_axis_name="c", subcore_axis_name="s")

    @functools.partial(
        pl.kernel,
        out_type=jax.ShapeDtypeStruct((NC * n_pad, d), jnp.float32),
        mesh=mesh,
        compiler_params=pltpu.CompilerParams(use_tc_tiling_on_sc=False),
        scratch_types=(
            [pltpu.VMEM((2, ch), jnp.int32) for _ in range(4)]
            + [pltpu.VMEM((ch, d), jnp.float32) for _ in range(4)]
            + [pltpu.VMEM_SHARED((n_pad, d), jnp.float32)]
            + [pltpu.SemaphoreType.DMA] * 12
        ),
    )
    def run(y_hbm, idx_hbm, z_hbm, out_hbm,
            ib0, ib1, ib2, ib3, db0, db1, db2, db3, accum,
            sl0, sl1, sl2, sl3, sg0, sg1, sg2, sg3, ss0, ss1, ss2, ss3):
        ib = [ib0, ib1, ib2, ib3]
        db = [db0, db1, db2, db3]
        sl = [sl0, sl1, sl2, sl3]
        sg = [sg0, sg1, sg2, sg3]
        ss = [ss0, ss1, ss2, ss3]

        c = lax.axis_index("c")
        s = lax.axis_index("s")
        tot = jnp.where(c == 0, cpt0, cpt1)
        base = jnp.where(c == 0, s * cpt0, NS * cpt0 + s * cpt1)

        pltpu.sync_copy(z_hbm.at[pl.ds(s * rpt, rpt)],
                        accum.at[pl.ds(s * rpt, rpt)])
        plsc.subcore_barrier()

        def load(i, k):
            pltpu.async_copy(idx_hbm.at[base + i], ib[k], sl[k])

        def load_wait(k):
            pltpu.make_async_copy(idx_hbm.at[0], ib[k], sl[k]).wait()

        def gather(i, k):
            del i
            pltpu.async_copy(y_hbm.at[ib[k].at[0]], db[k], sg[k])

        def gather_wait(k):
            pltpu.make_async_copy(y_hbm.at[ib[k].at[0]], db[k], sg[k]).wait()

        def scatter(i, k):
            del i
            pltpu.async_copy(db[k], accum.at[ib[k].at[1]], ss[k], add=True)

        def scatter_wait(k):
            pltpu.make_async_copy(db[k], accum.at[ib[k].at[1]], ss[k]).wait()

        load(0, 0)
        load(1, 1)
        load_wait(0)
        gather(0, 0)
        load(2, 2)
        load_wait(1)
        gather(1, 1)
        gather_wait(0)
        scatter(0, 0)
        load(3, 3)

        @pl.loop(0, (tot - 4) // 4)
        def _(q):
            i0 = 4 * q + 2
            for k in range(4):
                b, pr, f = (2 + k) % 4, (1 + k) % 4, k % 4
                load_wait(b)
                gather(i0 + k, b)
                gather_wait(pr)
                scatter(i0 + k - 1, pr)
                scatter_wait(f)
                load(i0 + k + 2, f)

        load_wait(2)
        gather(tot - 2, 2)
        gather_wait(1)
        scatter(tot - 3, 1)
        load_wait(3)
        gather(tot - 1, 3)
        gather_wait(2)
        scatter(tot - 2, 2)
        gather_wait(3)
        scatter(tot - 1, 3)
        for k in range(4):
            scatter_wait(k)

        plsc.subcore_barrier()
        pltpu.sync_copy(
            accum.at[pl.ds(s * rpt, rpt)],
            out_hbm.at[pl.ds(c * n_pad + s * rpt, rpt)],
        )

    return run(y, idx3d, zeros)


def _sc_degree(idx3d, n_pad, cpt0, cpt1):
    d = 16
    assert cpt0 % 4 == 0 and cpt1 % 4 == 0 and cpt0 >= 4 and cpt1 >= 4
    ch = idx3d.shape[2]
    rpt = n_pad // NS
    zeros = jnp.zeros((n_pad, d), jnp.float32)

    mesh = plsc.VectorSubcoreMesh(core_axis_name="c", subcore_axis_name="s")

    @functools.partial(
        pl.kernel,
        out_type=jax.ShapeDtypeStruct((NC * n_pad, d), jnp.float32),
        mesh=mesh,
        compiler_params=pltpu.CompilerParams(use_tc_tiling_on_sc=False),
        scratch_types=(
            [pltpu.VMEM((2, ch), jnp.int32) for _ in range(4)]
            + [pltpu.VMEM((ch, d), jnp.float32)]
            + [pltpu.VMEM_SHARED((n_pad, d), jnp.float32)]
            + [pltpu.SemaphoreType.DMA] * 8
        ),
    )
    def run(idx_hbm, z_hbm, out_hbm, ib0, ib1, ib2, ib3, ones, accum,
            sl0, sl1, sl2, sl3, ss0, ss1, ss2, ss3):
        ib = [ib0, ib1, ib2, ib3]
        sl = [sl0, sl1, sl2, sl3]
        ss = [ss0, ss1, ss2, ss3]

        c = lax.axis_index("c")
        s = lax.axis_index("s")
        tot = jnp.where(c == 0, cpt0, cpt1)
        base = jnp.where(c == 0, s * cpt0, NS * cpt0 + s * cpt1)

        @pl.loop(0, ch)
        def _(r):
            ones[r, pl.ds(0, 16)] = jnp.ones((16,), jnp.float32)

        pltpu.sync_copy(z_hbm.at[pl.ds(s * rpt, rpt)],
                        accum.at[pl.ds(s * rpt, rpt)])
        plsc.subcore_barrier()

        def load(i, k):
            pltpu.async_copy(idx_hbm.at[base + i], ib[k], sl[k])

        def load_wait(k):
            pltpu.make_async_copy(idx_hbm.at[0], ib[k], sl[k]).wait()

        def scatter(k):
            pltpu.async_copy(ones, accum.at[ib[k].at[1]], ss[k], add=True)

        def scatter_wait(k):
            pltpu.make_async_copy(ones, accum.at[ib[k].at[1]], ss[k]).wait()

        load(0, 0)
        load(1, 1)
        load_wait(0)
        scatter(0)
        load(2, 2)
        load_wait(1)
        scatter(1)
        load(3, 3)

        @pl.loop(0, (tot - 4) // 4)
        def _(q):
            i0 = 4 * q + 2
            for k in range(4):
                b, f = (2 + k) % 4, k % 4
                load_wait(b)
                scatter(b)
                scatter_wait(f)
                load(i0 + k + 2, f)

        load_wait(2)
        scatter(2)
        load_wait(3)
        scatter(3)
        for k in range(4):
            scatter_wait(k)

        plsc.subcore_barrier()
        pltpu.sync_copy(
            accum.at[pl.ds(s * rpt, rpt)],
            out_hbm.at[pl.ds(c * n_pad + s * rpt, rpt)],
        )

    return run(idx3d, zeros)


def _dinv(d0, d1):
    return lax.rsqrt(1.0 + d0[:, 0:1] + d1[:, 0:1])


def _mm_scale_body(x_ref, w_ref, d0_ref, d1_ref, xw_ref, y_ref):
    xw = jnp.dot(x_ref[...], w_ref[...],
                 preferred_element_type=jnp.float32,
                 precision=lax.Precision.HIGHEST)
    xw_ref[...] = xw
    y_ref[...] = xw * _dinv(d0_ref[...], d1_ref[...])


def _tc_matmul_scale(x_pad, w, deg_p):
    n_pad, k = x_pad.shape
    m = w.shape[1]
    nb = n_pad // RB
    return pl.pallas_call(
        _mm_scale_body,
        grid=(nb,),
        in_specs=[pl.BlockSpec((RB, k), lambda i: (i, 0)),
                  pl.BlockSpec((k, m), lambda i: (0, 0)),
                  pl.BlockSpec((RB, 16), lambda i: (i, 0)),
                  pl.BlockSpec((RB, 16), lambda i: (i + nb, 0))],
        out_specs=[pl.BlockSpec((RB, m), lambda i: (i, 0)),
                   pl.BlockSpec((RB, m), lambda i: (i, 0))],
        out_shape=[jax.ShapeDtypeStruct((n_pad, m), jnp.float32),
                   jax.ShapeDtypeStruct((n_pad, m), jnp.float32)],
    )(x_pad, w, deg_p, deg_p)


def _mid_body(d0, d1, a0, a1, xw1, b1, w2, xw2_o, y2_o):
    dinv = _dinv(d0[...], d1[...])
    h = (a0[...] + a1[...]) * dinv + xw1[...] * (dinv * dinv) + b1[...]
    h = jnp.maximum(h, 0.0)
    xw2 = jnp.dot(h, w2[...], preferred_element_type=jnp.float32,
                  precision=lax.Precision.HIGHEST)
    xw2_o[...] = xw2
    y2_o[...] = xw2 * dinv


def _tc_mid(deg_p, agg1_p, xw1, b1r, w2):
    n_pad, dh = xw1.shape
    do = w2.shape[1]
    nb = n_pad // RB
    return pl.pallas_call(
        _mid_body,
        grid=(nb,),
        in_specs=[pl.BlockSpec((RB, 16), lambda i: (i, 0)),
                  pl.BlockSpec((RB, 16), lambda i: (i + nb, 0)),
                  pl.BlockSpec((RB, dh), lambda i: (i, 0)),
                  pl.BlockSpec((RB, dh), lambda i: (i + nb, 0)),
                  pl.BlockSpec((RB, dh), lambda i: (i, 0)),
                  pl.BlockSpec((1, dh), lambda i: (0, 0)),
                  pl.BlockSpec((dh, do), lambda i: (0, 0))],
        out_specs=[pl.BlockSpec((RB, do), lambda i: (i, 0)),
                   pl.BlockSpec((RB, do), lambda i: (i, 0))],
        out_shape=[jax.ShapeDtypeStruct((n_pad, do), jnp.float32),
                   jax.ShapeDtypeStruct((n_pad, do), jnp.float32)],
    )(deg_p, deg_p, agg1_p, agg1_p, xw1, b1r, w2)


def _final_body(d0, d1, g0, g1, xw2, b2, o_ref):
    dinv = _dinv(d0[...], d1[...])
    o_ref[...] = (g0[...] + g1[...]) * dinv + xw2[...] * (dinv * dinv) + b2[...]


def _tc_final(deg_p, agg2_p, xw2, b2r, n):
    n_pad, do = xw2.shape
    rb = 1000
    nb = n // rb
    return pl.pallas_call(
        _final_body,
        grid=(nb,),
        in_specs=[pl.BlockSpec((rb, 16), lambda i: (i, 0)),
                  pl.BlockSpec((rb, 16), lambda i: (i, 0)),
                  pl.BlockSpec((rb, do), lambda i: (i, 0)),
                  pl.BlockSpec((rb, do), lambda i: (i, 0)),
                  pl.BlockSpec((rb, do), lambda i: (i, 0)),
                  pl.BlockSpec((1, do), lambda i: (0, 0))],
        out_specs=pl.BlockSpec((rb, do), lambda i: (i, 0)),
        out_shape=jax.ShapeDtypeStruct((n, do), jnp.float32),
    )(deg_p[:n_pad], deg_p[n_pad:], agg2_p[:n_pad], agg2_p[n_pad:], xw2, b2r)


def kernel(x, edge_index, W1, b1, W2, b2):
    n, d_in = x.shape
    e = edge_index.shape[1]

    blk = NS * CH
    n_pad = ((n + 1 + blk - 1) // blk) * blk

    def make_idx(ch, cpt0, cpt1):
        n_chunks = NS * (cpt0 + cpt1)
        e_pad = n_chunks * ch
        pad = jnp.full((e_pad - e,), n, dtype=jnp.int32)
        s2 = jnp.concatenate([edge_index[0], pad]).reshape(n_chunks, 1, ch)
        d2 = jnp.concatenate([edge_index[1], pad]).reshape(n_chunks, 1, ch)
        return jnp.concatenate([s2, d2], axis=1)

    C0_DEG, C1_DEG = 96, 64
    C0_A2, C1_A2 = 132, 28
    C0_64, C1_64 = 288, 28
    idx128 = make_idx(CH, C0_DEG, C1_DEG)
    idx64 = make_idx(64, C0_64, C1_64)

    x_pad = jnp.pad(x, ((0, n_pad - n), (0, 0)))
    deg_p = _sc_degree(idx128, n_pad, C0_DEG, C1_DEG)
    xw1, y1 = _tc_matmul_scale(x_pad, W1, deg_p)
    agg1_p = _sc_segment_sum(y1, idx64, n_pad, 64, C0_64, C1_64)
    xw2, y2 = _tc_mid(deg_p, agg1_p, xw1, b1.reshape(1, -1), W2)
    agg2_p = _sc_segment_sum(y2, idx128, n_pad, CH, C0_A2, C1_A2)
    return _tc_final(deg_p, agg2_p, xw2, b2.reshape(1, -1), n)
